# baseline (speedup 1.0000x reference)
import functools

import jax
import jax.numpy as jnp
from jax import lax
from jax.experimental import pallas as pl
from jax.experimental.pallas import tpu as pltpu
from jax.experimental.pallas import tpu_sc as plsc

WORD_DIM = 64
IDX_MINOR = 128
NUM_CORES = 2
NUM_SUBCORES = 16
NUM_WORKERS = NUM_CORES * NUM_SUBCORES
CHUNK = 8


def _emb_body(rows_per_w, x_hbm, table_hbm, out_hbm, idx_v, rows_v, sem):
    wid = lax.axis_index("s") * NUM_CORES + lax.axis_index("c")
    base = wid * rows_per_w

    def chunk_step(i, carry):
        r0 = base + i * CHUNK
        pltpu.sync_copy(x_hbm.at[pl.ds(r0, CHUNK)], idx_v)
        cps = [
            pltpu.async_copy(table_hbm.at[idx_v.at[j]], rows_v.at[j], sem)
            for j in range(CHUNK)
        ]
        for cp in cps:
            cp.wait()
        pltpu.sync_copy(rows_v, out_hbm.at[pl.ds(r0, CHUNK)])
        return carry

    lax.fori_loop(0, rows_per_w // CHUNK, chunk_step, 0)


def kernel(x, emb_table):
    batch, seq = x.shape
    total = batch * seq
    assert total % (NUM_WORKERS * IDX_MINOR * CHUNK) == 0
    n_rows = total // IDX_MINOR
    rows_per_w = n_rows // NUM_WORKERS

    x_flat = x.reshape(n_rows, IDX_MINOR).astype(jnp.int32)

    mesh = plsc.VectorSubcoreMesh(core_axis_name="c", subcore_axis_name="s")
    emb_kernel = pl.kernel(
        functools.partial(_emb_body, rows_per_w),
        out_type=jax.ShapeDtypeStruct((n_rows, IDX_MINOR, WORD_DIM),
                                      emb_table.dtype),
        mesh=mesh,
        scratch_types=[
            pltpu.VMEM((CHUNK, IDX_MINOR), jnp.int32),
            pltpu.VMEM((CHUNK, IDX_MINOR, WORD_DIM), emb_table.dtype),
            pltpu.SemaphoreType.DMA,
        ],
        compiler_params=pltpu.CompilerParams(use_tc_tiling_on_sc=False),
    )
    out = emb_kernel(x_flat, emb_table)
    return out.reshape(batch, seq, WORD_DIM)


# speedup vs baseline: 1.0139x; 1.0139x over previous
import functools

import jax
import jax.numpy as jnp
from jax import lax
from jax.experimental import pallas as pl
from jax.experimental.pallas import tpu as pltpu
from jax.experimental.pallas import tpu_sc as plsc

WORD_DIM = 64
IDX_MINOR = 128
NUM_CORES = 2
NUM_SUBCORES = 16
NUM_WORKERS = NUM_CORES * NUM_SUBCORES
CHUNK = 5


def _emb_body(rows_per_w, x_hbm, table_hbm, out_hbm, idx_v, rows_v,
              gsem0, gsem1, ssem0, ssem1):
    n_chunks = rows_per_w // CHUNK
    gsems = (gsem0, gsem1)
    ssems = (ssem0, ssem1)
    wid = lax.axis_index("s") * NUM_CORES + lax.axis_index("c")
    base = wid * rows_per_w

    pltpu.sync_copy(x_hbm.at[pl.ds(base, rows_per_w)], idx_v)

    def fire_gather(b, c):
        for j in range(CHUNK):
            pltpu.async_copy(table_hbm.at[idx_v.at[c * CHUNK + j]],
                             rows_v.at[b, j], gsems[b])

    def drain_gather(b):
        for j in range(CHUNK):
            pltpu.make_async_copy(table_hbm.at[idx_v.at[j]],
                                  rows_v.at[b, j], gsems[b]).wait()

    def fire_store(b, c):
        pltpu.async_copy(rows_v.at[b],
                         out_hbm.at[pl.ds(base + c * CHUNK, CHUNK)], ssems[b])

    def wait_store(b):
        pltpu.make_async_copy(rows_v.at[b],
                              out_hbm.at[pl.ds(base, CHUNK)], ssems[b]).wait()

    fire_gather(0, 0)

    def pair_step(gg, carry):
        c0 = 2 * gg
        c1 = c0 + 1
        drain_gather(0)

        @pl.when(gg > 0)
        def _():
            wait_store(1)

        fire_gather(1, c1)
        fire_store(0, c0)
        drain_gather(1)
        wait_store(0)

        @pl.when(gg + 1 < n_chunks // 2)
        def _():
            fire_gather(0, c0 + 2)

        fire_store(1, c1)
        return carry

    lax.fori_loop(0, n_chunks // 2, pair_step, 0)
    wait_store(1)


def kernel(x, emb_table):
    batch, seq = x.shape
    total = batch * seq
    assert total % (NUM_WORKERS * IDX_MINOR * CHUNK * 2) == 0
    n_rows = total // IDX_MINOR
    rows_per_w = n_rows // NUM_WORKERS

    x_flat = x.reshape(n_rows, IDX_MINOR).astype(jnp.int32)

    mesh = plsc.VectorSubcoreMesh(core_axis_name="c", subcore_axis_name="s")
    emb_kernel = pl.kernel(
        functools.partial(_emb_body, rows_per_w),
        out_type=jax.ShapeDtypeStruct((n_rows, IDX_MINOR, WORD_DIM),
                                      emb_table.dtype),
        mesh=mesh,
        scratch_types=[
            pltpu.VMEM((rows_per_w, IDX_MINOR), jnp.int32),
            pltpu.VMEM((2, CHUNK, IDX_MINOR, WORD_DIM), emb_table.dtype),
            pltpu.SemaphoreType.DMA,
            pltpu.SemaphoreType.DMA,
            pltpu.SemaphoreType.DMA,
            pltpu.SemaphoreType.DMA,
        ],
        compiler_params=pltpu.CompilerParams(use_tc_tiling_on_sc=False),
    )
    out = emb_kernel(x_flat, emb_table)
    return out.reshape(batch, seq, WORD_DIM)
